# baseline (speedup 1.0000x reference)
import functools

import jax
import jax.numpy as jnp
from jax import lax
from jax.experimental import pallas as pl
from jax.experimental.pallas import tpu as pltpu
from jax.experimental.pallas import tpu_sc as plsc

B = 16
L = 2048
D = 256

NC = 2
NS = 16
SEG = 2 * L + 1
CHUNK = 64
ROWS0 = 1088
ROWS1 = L - ROWS0
NBUF = 4


def _sc_preprocess(item_values, action_values, contextual_values):
    mesh = plsc.VectorSubcoreMesh(core_axis_name="c", subcore_axis_name="s")
    out_rows = B * SEG
    nt_max = 2 * (ROWS0 // CHUNK)

    @functools.partial(
        pl.kernel,
        mesh=mesh,
        out_type=(
            jax.ShapeDtypeStruct((out_rows, D), jnp.float32),
            jax.ShapeDtypeStruct((B,), jnp.int32),
            jax.ShapeDtypeStruct((B + 1,), jnp.int32),
        ),
        scratch_types=(
            [pltpu.VMEM((CHUNK, D), jnp.float32) for _ in range(NBUF)]
            + [pltpu.VMEM((CHUNK,), jnp.int32) for _ in range(nt_max)]
            + [pltpu.VMEM((1, D), jnp.float32)]
            + [pltpu.VMEM((16,), jnp.int32), pltpu.VMEM((32,), jnp.int32)]
            + [pltpu.SemaphoreType.DMA for _ in range(2 * NBUF + 3)]
        ),
    )
    def k(item_hbm, action_hbm, ctx_hbm, out_hbm, len_hbm, off_hbm, *scr):
        bufs = scr[0:NBUF]
        idxs = scr[NBUF:NBUF + nt_max]
        base_i = NBUF + nt_max
        ctx_buf = scr[base_i]
        len_buf = scr[base_i + 1]
        off_buf = scr[base_i + 2]
        gsem = scr[base_i + 3:base_i + 3 + NBUF]
        ssem = scr[base_i + 3 + NBUF:base_i + 3 + 2 * NBUF]
        csem_g = scr[base_i + 3 + 2 * NBUF]
        csem_s = scr[base_i + 3 + 2 * NBUF + 1]
        msem = scr[base_i + 3 + 2 * NBUF + 2]
        sid = lax.axis_index("s")
        core = lax.axis_index("c")
        wid = sid * NC + core
        b = wid // 2
        lane = lax.iota(jnp.int32, 16)

        is_ctx = wid < B

        @pl.when(is_ctx)
        def _():
            pltpu.async_copy(ctx_hbm.at[pl.ds(wid, 1)], ctx_buf,
                             csem_g).wait()
            pltpu.async_copy(ctx_buf, out_hbm.at[pl.ds(wid * SEG, 1)],
                             csem_s)

        @pl.when(wid == 0)
        def _():
            len_buf[...] = lane * 0 + SEG
            off_buf[pl.ds(0, 16)] = SEG * lane
            off_buf[pl.ds(16, 16)] = SEG * (16 + lane)
            pltpu.async_copy(len_buf, len_hbm, msem)
            pltpu.async_copy(off_buf.at[pl.ds(0, B + 1)], off_hbm, msem)

        def emit_main(row_off, nch):
            nt = 2 * nch
            src_base = b * L + row_off
            out_base = b * SEG + 1 + 2 * row_off

            def start_gather(t):
                ref = item_hbm if t % 2 == 0 else action_hbm
                src = src_base + (t // 2) * CHUNK
                return pltpu.async_copy(ref.at[pl.ds(src, CHUNK)],
                                        bufs[t % NBUF], gsem[t % NBUF])

            gath = {t: start_gather(t) for t in range(min(2, nt))}
            for t in range(nt):
                base = out_base + (t % 2) + 2 * (t // 2) * CHUNK
                for i in range(CHUNK // 16):
                    idxs[t][pl.ds(i * 16, 16)] = base + 2 * (i * 16 + lane)
            scat = {}
            for t in range(nt):
                if t + 2 < nt:
                    if t >= 2:
                        scat[t - 2].wait()
                    gath[t + 2] = start_gather(t + 2)
                gath[t].wait()
                scat[t] = pltpu.async_copy(bufs[t % NBUF],
                                           out_hbm.at[idxs[t]],
                                           ssem[t % NBUF])
            for t in range(max(0, nt - NBUF), nt):
                scat[t].wait()

        @pl.when(core == 0)
        def _():
            emit_main(0, ROWS0 // CHUNK)

        @pl.when(core == 1)
        def _():
            emit_main(ROWS0, ROWS1 // CHUNK)

        @pl.when(is_ctx)
        def _():
            pltpu.make_async_copy(ctx_buf, out_hbm.at[pl.ds(wid * SEG, 1)],
                                  csem_s).wait()

        @pl.when(wid == 0)
        def _():
            pltpu.make_async_copy(len_buf, len_hbm, msem).wait()
            pltpu.make_async_copy(off_buf.at[pl.ds(0, B + 1)], off_hbm,
                                  msem).wait()

    return k(item_values, action_values, contextual_values)


def kernel(item_values, action_values, contextual_values):
    out_values, out_lengths, out_offsets = _sc_preprocess(
        item_values, action_values, contextual_values)
    return out_values, out_lengths, out_offsets
